# Initial kernel scaffold; baseline (speedup 1.0000x reference)
import jax
import jax.numpy as jnp
from jax.experimental import pallas as pl


def kernel(x, edge_index, edge_attr, batch, W_emb, b_emb, Wf0, bf0, Ws0, bs0, g0, be0, Wf1, bf1, Ws1, bs1, g1, be1, Wf2, bf2, Ws2, bs2, g2, be2, W_fc, b_fc, Wh0, bh0, Wh1, bh1, W_out, b_out):
    raise NotImplementedError("write your pallas kernel here")


# speedup vs baseline: 1.7395x; 1.7395x over previous
import functools

import jax
import jax.numpy as jnp
from jax import lax
from jax.experimental import pallas as pl
from jax.experimental.pallas import tpu as pltpu
from jax.experimental.pallas import tpu_sc as plsc

N = 10000
E = 320000
D_IN = 128
A = 64
EF = 64
H = 128
G = 64

NC = 2
NS = 16
NW = NC * NS
EPW = E // NW
C = 80
NCH = EPW // C
NPAD = 10240
RPS = NPAD // NS

_F32 = jnp.float32


def _emb_body(x_ref, wemb_ref, bemb_ref, wn_ref, bn_ref, h_ref, pd_ref, ps_ref):
    h = jnp.dot(x_ref[...], wemb_ref[...], preferred_element_type=_F32) + bemb_ref[...]
    h_ref[...] = h
    p = jnp.dot(h, wn_ref[...], preferred_element_type=_F32) + bn_ref[...]
    pd_ref[...] = p[:, :2 * A]
    ps_ref[...] = p[:, 2 * A:]


def _embed_project(x, wembT, bemb, wn, bn):
    return pl.pallas_call(
        _emb_body,
        out_shape=(
            jax.ShapeDtypeStruct((N, A), _F32),
            jax.ShapeDtypeStruct((N, 2 * A), _F32),
            jax.ShapeDtypeStruct((N, 2 * A), _F32),
        ),
    )(x, wembT, bemb, wn, bn)


_GAMMA = (1.0 / (8.0 / (EF - 1)) ** 2)
_STEP = 8.0 / (EF - 1)

BLK_E = 4000


def _edge_body(d_ref, zd_ref, zs_ref, we_ref, out_ref):
    d = d_ref[...]
    centers = lax.broadcasted_iota(jnp.int32, (1, EF), 1).astype(_F32) * _STEP
    ea = jnp.exp(-_GAMMA * (d - centers) ** 2)
    t = zd_ref[...] + zs_ref[...]
    t = t + jnp.dot(ea, we_ref[...], preferred_element_type=_F32)
    a = t[:, :A]
    b = t[:, A:]
    out_ref[...] = jax.nn.sigmoid(a) * jax.nn.softplus(b)


def _edge_stage(d, zd, zs, we):
    grid = (E // BLK_E,)
    return pl.pallas_call(
        _edge_body,
        grid=grid,
        in_specs=[
            pl.BlockSpec((BLK_E, 1), lambda i: (i, 0)),
            pl.BlockSpec((BLK_E, 2 * A), lambda i: (i, 0)),
            pl.BlockSpec((BLK_E, 2 * A), lambda i: (i, 0)),
            pl.BlockSpec((EF, 2 * A), lambda i: (0, 0)),
        ],
        out_specs=pl.BlockSpec((BLK_E, A), lambda i: (i, 0)),
        out_shape=jax.ShapeDtypeStruct((E, A), _F32),
    )(d, zd, zs, we)


def _bn_proj_body(parts_ref, h_ref, g_ref, be_ref, wn_ref, bn_ref,
                  hout_ref, pd_ref, ps_ref):
    t = parts_ref[0, :N] + parts_ref[1, :N] + h_ref[...]
    mu = jnp.mean(t, axis=0, keepdims=True)
    var = jnp.mean((t - mu) ** 2, axis=0, keepdims=True)
    hn = g_ref[...] * (t - mu) / jnp.sqrt(var + 1e-5) + be_ref[...]
    hout_ref[...] = hn
    p = jnp.dot(hn, wn_ref[...], preferred_element_type=_F32) + bn_ref[...]
    pd_ref[...] = p[:, :2 * A]
    ps_ref[...] = p[:, 2 * A:]


def _bn_project(parts, h, g, be, wn, bn):
    return pl.pallas_call(
        _bn_proj_body,
        out_shape=(
            jax.ShapeDtypeStruct((N, A), _F32),
            jax.ShapeDtypeStruct((N, 2 * A), _F32),
            jax.ShapeDtypeStruct((N, 2 * A), _F32),
        ),
    )(parts, h, g, be, wn, bn)


def _head_body(parts_ref, h_ref, g_ref, be_ref, batch_ref,
               wfc_ref, bfc_ref, wh0_ref, bh0_ref, wh1_ref, bh1_ref,
               wout_ref, bout_ref, out_ref):
    t = parts_ref[0, :N] + parts_ref[1, :N] + h_ref[...]
    mu = jnp.mean(t, axis=0, keepdims=True)
    var = jnp.mean((t - mu) ** 2, axis=0, keepdims=True)
    hn = g_ref[...] * (t - mu) / jnp.sqrt(var + 1e-5) + be_ref[...]

    seg = lax.broadcasted_iota(jnp.int32, (1, G), 1)
    mask = (batch_ref[...] == seg).astype(_F32)
    sums = lax.dot_general(mask, hn, (((0,), (0,)), ((), ())),
                           preferred_element_type=_F32)
    ones = jnp.ones((N, 1), _F32)
    cnt = lax.dot_general(mask, ones, (((0,), (0,)), ((), ())),
                          preferred_element_type=_F32)
    pooled = sums / jnp.maximum(cnt, 1.0)

    o = jax.nn.softplus(pooled)
    o = jax.nn.softplus(jnp.dot(o, wfc_ref[...], preferred_element_type=_F32) + bfc_ref[...])
    o = jax.nn.softplus(jnp.dot(o, wh0_ref[...], preferred_element_type=_F32) + bh0_ref[...])
    o = jax.nn.softplus(jnp.dot(o, wh1_ref[...], preferred_element_type=_F32) + bh1_ref[...])
    out_ref[...] = jnp.dot(o, wout_ref[...], preferred_element_type=_F32) + bout_ref[...]


def _head(parts, h, g, be, batch2, wfcT, bfc, wh0T, bh0, wh1T, bh1, woutT, bout):
    return pl.pallas_call(
        _head_body,
        out_shape=jax.ShapeDtypeStruct((G, 1), _F32),
    )(parts, h, g, be, batch2, wfcT, bfc, wh0T, bh0, wh1T, bh1, woutT, bout)


def _gather_body(pd_hbm, ps_hbm, dst_hbm, src_hbm, zd_hbm, zs_hbm,
                 idx_d, idx_s, bufd, bufs, sem):
    wid = lax.axis_index("s") * NC + lax.axis_index("c")
    e_base = wid * EPW

    def chunk(ci, carry):
        e0 = e_base + ci * C
        pltpu.sync_copy(dst_hbm.at[pl.ds(e0, C)], idx_d)
        pltpu.sync_copy(src_hbm.at[pl.ds(e0, C)], idx_s)
        pltpu.async_copy(pd_hbm.at[idx_d], bufd, sem).wait()
        pltpu.async_copy(ps_hbm.at[idx_s], bufs, sem).wait()
        pltpu.sync_copy(bufd, zd_hbm.at[pl.ds(e0, C)])
        pltpu.sync_copy(bufs, zs_hbm.at[pl.ds(e0, C)])
        return carry

    lax.fori_loop(0, NCH, chunk, 0)


@functools.cache
def _sc_gather_kernel():
    mesh = plsc.VectorSubcoreMesh(core_axis_name="c", subcore_axis_name="s",
                                  num_cores=NC, num_subcores=NS)
    return pl.kernel(
        _gather_body,
        out_type=(
            jax.ShapeDtypeStruct((E, 2 * A), _F32),
            jax.ShapeDtypeStruct((E, 2 * A), _F32),
        ),
        mesh=mesh,
        scratch_types=[
            pltpu.VMEM((C,), jnp.int32),
            pltpu.VMEM((C,), jnp.int32),
            pltpu.VMEM((C, 2 * A), _F32),
            pltpu.VMEM((C, 2 * A), _F32),
            pltpu.SemaphoreType.DMA,
        ],
    )


def _sc_gather(pd, ps, dst2, src2):
    return _sc_gather_kernel()(pd, ps, dst2, src2)


def _scatter_body(contrib_hbm, dst_hbm, zeros_hbm, out_hbm, idxb, buf, aggr_sh, sem):
    cid = lax.axis_index("c")
    sid = lax.axis_index("s")
    wid = sid * NC + cid

    pltpu.sync_copy(zeros_hbm.at[pl.ds(sid * RPS, RPS)],
                    aggr_sh.at[pl.ds(sid * RPS, RPS)])
    plsc.subcore_barrier()

    def chunk(ci, carry):
        e0 = ci * C
        pltpu.sync_copy(dst_hbm.at[pl.ds(e0, C)], idxb)
        pltpu.sync_copy(contrib_hbm.at[pl.ds(e0, C)], buf)
        pltpu.sync_copy(buf, aggr_sh.at[idxb], add=True)
        return carry

    @pl.when(jnp.logical_and(sid == 0, cid == 0))
    def _():
        lax.fori_loop(0, E // C, chunk, 0)
    plsc.subcore_barrier()

    pltpu.sync_copy(aggr_sh.at[pl.ds(sid * RPS, RPS)],
                    out_hbm.at[cid, pl.ds(sid * RPS, RPS)])


@functools.cache
def _sc_scatter_kernel():
    mesh = plsc.VectorSubcoreMesh(core_axis_name="c", subcore_axis_name="s",
                                  num_cores=NC, num_subcores=NS)
    return pl.kernel(
        _scatter_body,
        out_type=jax.ShapeDtypeStruct((NC, NPAD, A), _F32),
        mesh=mesh,
        scratch_types=[
            pltpu.VMEM((C,), jnp.int32),
            pltpu.VMEM((C, A), _F32),
            pltpu.VMEM_SHARED((NPAD, A), _F32),
            pltpu.SemaphoreType.DMA,
        ],
    )


def _sc_scatter(contrib, dst2, zeros_na):
    p0 = jnp.zeros((NPAD, A), _F32).at[dst2].add(contrib)
    return jnp.stack([p0, jnp.zeros_like(p0)])


def kernel(x, edge_index, edge_attr, batch, W_emb, b_emb,
           Wf0, bf0, Ws0, bs0, g0, be0,
           Wf1, bf1, Ws1, bs1, g1, be1,
           Wf2, bf2, Ws2, bs2, g2, be2,
           W_fc, b_fc, Wh0, bh0, Wh1, bh1, W_out, b_out):
    dst2 = edge_index[1].reshape(E)
    src2 = edge_index[0].reshape(E)
    d2 = edge_attr.reshape(E, 1)
    batch2 = batch.reshape(N, 1)
    zeros_na = jnp.zeros((NPAD, A), _F32)

    def node_w(Wf, bf, Ws, bs):
        wn = jnp.concatenate(
            [Wf[:, :A].T, Ws[:, :A].T, Wf[:, A:2 * A].T, Ws[:, A:2 * A].T], axis=1)
        bn = jnp.concatenate([bf, bs, jnp.zeros((2 * A,), _F32)]).reshape(1, 4 * A)
        we = jnp.concatenate([Wf[:, 2 * A:].T, Ws[:, 2 * A:].T], axis=1)
        return wn, bn, we

    wn0, bn0, we0 = node_w(Wf0, bf0, Ws0, bs0)
    wn1, bn1, we1 = node_w(Wf1, bf1, Ws1, bs1)
    wn2, bn2, we2 = node_w(Wf2, bf2, Ws2, bs2)

    h, pd, ps = _embed_project(x, W_emb.T, b_emb.reshape(1, A), wn0, bn0)

    lw = [(we0, g0, be0, wn1, bn1), (we1, g1, be1, wn2, bn2)]
    for we, g, be, wn_n, bn_n in lw:
        zd, zs = _sc_gather(pd, ps, dst2, src2)
        contrib = _edge_stage(d2, zd, zs, we)
        parts = _sc_scatter(contrib, dst2, zeros_na)
        h, pd, ps = _bn_project(parts, h, g.reshape(1, A), be.reshape(1, A), wn_n, bn_n)

    zd, zs = _sc_gather(pd, ps, dst2, src2)
    contrib = _edge_stage(d2, zd, zs, we2)
    parts = _sc_scatter(contrib, dst2, zeros_na)

    out = _head(parts, h, g2.reshape(1, A), be2.reshape(1, A), batch2,
                W_fc.T, b_fc.reshape(1, H),
                Wh0.T, bh0.reshape(1, H), Wh1.T, bh1.reshape(1, H),
                W_out.T, b_out.reshape(1, 1))
    return out


# speedup vs baseline: 1.7511x; 1.0067x over previous
import functools

import jax
import jax.numpy as jnp
from jax import lax
from jax.experimental import pallas as pl
from jax.experimental.pallas import tpu as pltpu
from jax.experimental.pallas import tpu_sc as plsc

N = 10000
E = 320000
D_IN = 128
A = 64
EF = 64
H = 128
G = 64

NC = 2
NS = 16
NW = NC * NS
EPW = E // NW
C = 80
NCH = EPW // C
NPAD = 10240
RPS = NPAD // NS

_F32 = jnp.float32


def _emb_body(x_ref, wemb_ref, bemb_ref, wn_ref, bn_ref, h_ref, pd_ref, ps_ref):
    h = jnp.dot(x_ref[...], wemb_ref[...], preferred_element_type=_F32) + bemb_ref[...]
    h_ref[...] = h
    p = jnp.dot(h, wn_ref[...], preferred_element_type=_F32) + bn_ref[...]
    pd_ref[...] = p[:, :2 * A]
    ps_ref[...] = p[:, 2 * A:]


def _embed_project(x, wembT, bemb, wn, bn):
    return pl.pallas_call(
        _emb_body,
        out_shape=(
            jax.ShapeDtypeStruct((N, A), _F32),
            jax.ShapeDtypeStruct((N, 2 * A), _F32),
            jax.ShapeDtypeStruct((N, 2 * A), _F32),
        ),
    )(x, wembT, bemb, wn, bn)


_GAMMA = (1.0 / (8.0 / (EF - 1)) ** 2)
_STEP = 8.0 / (EF - 1)

BLK_E = 4000


def _edge_body(d_ref, zd_ref, zs_ref, we_ref, out_ref):
    d = d_ref[...]
    centers = lax.broadcasted_iota(jnp.int32, (1, EF), 1).astype(_F32) * _STEP
    ea = jnp.exp(-_GAMMA * (d - centers) ** 2)
    t = zd_ref[...] + zs_ref[...]
    t = t + jnp.dot(ea, we_ref[...], preferred_element_type=_F32)
    a = t[:, :A]
    b = t[:, A:]
    out_ref[...] = jax.nn.sigmoid(a) * jax.nn.softplus(b)


def _edge_stage(d, zd, zs, we):
    grid = (E // BLK_E,)
    return pl.pallas_call(
        _edge_body,
        grid=grid,
        in_specs=[
            pl.BlockSpec((BLK_E, 1), lambda i: (i, 0)),
            pl.BlockSpec((BLK_E, 2 * A), lambda i: (i, 0)),
            pl.BlockSpec((BLK_E, 2 * A), lambda i: (i, 0)),
            pl.BlockSpec((EF, 2 * A), lambda i: (0, 0)),
        ],
        out_specs=pl.BlockSpec((BLK_E, A), lambda i: (i, 0)),
        out_shape=jax.ShapeDtypeStruct((E, A), _F32),
    )(d, zd, zs, we)


def _bn_proj_body(parts_ref, h_ref, g_ref, be_ref, wn_ref, bn_ref,
                  hout_ref, pd_ref, ps_ref):
    t = parts_ref[...] + h_ref[...]
    mu = jnp.mean(t, axis=0, keepdims=True)
    var = jnp.mean((t - mu) ** 2, axis=0, keepdims=True)
    hn = g_ref[...] * (t - mu) / jnp.sqrt(var + 1e-5) + be_ref[...]
    hout_ref[...] = hn
    p = jnp.dot(hn, wn_ref[...], preferred_element_type=_F32) + bn_ref[...]
    pd_ref[...] = p[:, :2 * A]
    ps_ref[...] = p[:, 2 * A:]


def _bn_project(parts, h, g, be, wn, bn):
    return pl.pallas_call(
        _bn_proj_body,
        out_shape=(
            jax.ShapeDtypeStruct((N, A), _F32),
            jax.ShapeDtypeStruct((N, 2 * A), _F32),
            jax.ShapeDtypeStruct((N, 2 * A), _F32),
        ),
    )(parts, h, g, be, wn, bn)


def _head_body(parts_ref, h_ref, g_ref, be_ref, batch_ref,
               wfc_ref, bfc_ref, wh0_ref, bh0_ref, wh1_ref, bh1_ref,
               wout_ref, bout_ref, out_ref):
    t = parts_ref[...] + h_ref[...]
    mu = jnp.mean(t, axis=0, keepdims=True)
    var = jnp.mean((t - mu) ** 2, axis=0, keepdims=True)
    hn = g_ref[...] * (t - mu) / jnp.sqrt(var + 1e-5) + be_ref[...]

    seg = lax.broadcasted_iota(jnp.int32, (1, G), 1)
    mask = (batch_ref[...] == seg).astype(_F32)
    sums = lax.dot_general(mask, hn, (((0,), (0,)), ((), ())),
                           preferred_element_type=_F32)
    ones = jnp.ones((N, 1), _F32)
    cnt = lax.dot_general(mask, ones, (((0,), (0,)), ((), ())),
                          preferred_element_type=_F32)
    pooled = sums / jnp.maximum(cnt, 1.0)

    o = jax.nn.softplus(pooled)
    o = jax.nn.softplus(jnp.dot(o, wfc_ref[...], preferred_element_type=_F32) + bfc_ref[...])
    o = jax.nn.softplus(jnp.dot(o, wh0_ref[...], preferred_element_type=_F32) + bh0_ref[...])
    o = jax.nn.softplus(jnp.dot(o, wh1_ref[...], preferred_element_type=_F32) + bh1_ref[...])
    out_ref[...] = jnp.dot(o, wout_ref[...], preferred_element_type=_F32) + bout_ref[...]


def _head(parts, h, g, be, batch2, wfcT, bfc, wh0T, bh0, wh1T, bh1, woutT, bout):
    return pl.pallas_call(
        _head_body,
        out_shape=jax.ShapeDtypeStruct((G, 1), _F32),
    )(parts, h, g, be, batch2, wfcT, bfc, wh0T, bh0, wh1T, bh1, woutT, bout)


def _gather_body(pd_hbm, ps_hbm, dst_hbm, src_hbm, zd_hbm, zs_hbm,
                 idx_d, idx_s, bufd, bufs, sem):
    wid = lax.axis_index("s") * NC + lax.axis_index("c")
    e_base = wid * EPW

    def chunk(ci, carry):
        e0 = e_base + ci * C
        pltpu.sync_copy(dst_hbm.at[pl.ds(e0, C)], idx_d)
        pltpu.sync_copy(src_hbm.at[pl.ds(e0, C)], idx_s)
        pltpu.async_copy(pd_hbm.at[idx_d], bufd, sem).wait()
        pltpu.async_copy(ps_hbm.at[idx_s], bufs, sem).wait()
        pltpu.sync_copy(bufd, zd_hbm.at[pl.ds(e0, C)])
        pltpu.sync_copy(bufs, zs_hbm.at[pl.ds(e0, C)])
        return carry

    lax.fori_loop(0, NCH, chunk, 0)


@functools.cache
def _sc_gather_kernel():
    mesh = plsc.VectorSubcoreMesh(core_axis_name="c", subcore_axis_name="s",
                                  num_cores=NC, num_subcores=NS)
    return pl.kernel(
        _gather_body,
        out_type=(
            jax.ShapeDtypeStruct((E, 2 * A), _F32),
            jax.ShapeDtypeStruct((E, 2 * A), _F32),
        ),
        mesh=mesh,
        scratch_types=[
            pltpu.VMEM((C,), jnp.int32),
            pltpu.VMEM((C,), jnp.int32),
            pltpu.VMEM((C, 2 * A), _F32),
            pltpu.VMEM((C, 2 * A), _F32),
            pltpu.SemaphoreType.DMA,
        ],
    )


def _sc_gather(pd, ps, dst2, src2):
    return _sc_gather_kernel()(pd, ps, dst2, src2)


def kernel(x, edge_index, edge_attr, batch, W_emb, b_emb,
           Wf0, bf0, Ws0, bs0, g0, be0,
           Wf1, bf1, Ws1, bs1, g1, be1,
           Wf2, bf2, Ws2, bs2, g2, be2,
           W_fc, b_fc, Wh0, bh0, Wh1, bh1, W_out, b_out):
    dst2 = edge_index[1].reshape(E)
    src2 = edge_index[0].reshape(E)
    d2 = edge_attr.reshape(E, 1)
    batch2 = batch.reshape(N, 1)

    def node_w(Wf, bf, Ws, bs):
        wn = jnp.concatenate(
            [Wf[:, :A].T, Ws[:, :A].T, Wf[:, A:2 * A].T, Ws[:, A:2 * A].T], axis=1)
        bn = jnp.concatenate([bf, bs, jnp.zeros((2 * A,), _F32)]).reshape(1, 4 * A)
        we = jnp.concatenate([Wf[:, 2 * A:].T, Ws[:, 2 * A:].T], axis=1)
        return wn, bn, we

    wn0, bn0, we0 = node_w(Wf0, bf0, Ws0, bs0)
    wn1, bn1, we1 = node_w(Wf1, bf1, Ws1, bs1)
    wn2, bn2, we2 = node_w(Wf2, bf2, Ws2, bs2)

    h, pd, ps = _embed_project(x, W_emb.T, b_emb.reshape(1, A), wn0, bn0)

    lw = [(we0, g0, be0, wn1, bn1), (we1, g1, be1, wn2, bn2)]
    for we, g, be, wn_n, bn_n in lw:
        zd, zs = _sc_gather(pd, ps, dst2, src2)
        contrib = _edge_stage(d2, zd, zs, we)
        parts = jax.ops.segment_sum(contrib, dst2, num_segments=N)
        h, pd, ps = _bn_project(parts, h, g.reshape(1, A), be.reshape(1, A), wn_n, bn_n)

    zd, zs = _sc_gather(pd, ps, dst2, src2)
    contrib = _edge_stage(d2, zd, zs, we2)
    parts = jax.ops.segment_sum(contrib, dst2, num_segments=N)

    out = _head(parts, h, g2.reshape(1, A), be2.reshape(1, A), batch2,
                W_fc.T, b_fc.reshape(1, H),
                Wh0.T, bh0.reshape(1, H), Wh1.T, bh1.reshape(1, H),
                W_out.T, b_out.reshape(1, 1))
    return out


# speedup vs baseline: 1.8952x; 1.0823x over previous
import functools

import jax
import jax.numpy as jnp
from jax import lax
from jax.experimental import pallas as pl
from jax.experimental.pallas import tpu as pltpu
from jax.experimental.pallas import tpu_sc as plsc

N = 10000
E = 320000
D_IN = 128
A = 64
EF = 64
H = 128
G = 64

NC = 2
NS = 16
NW = NC * NS
EPW = E // NW
C = 80
NCH = EPW // C
NPAD = 10240
RPS = NPAD // NS

_F32 = jnp.float32


def _emb_body(x_ref, wemb_ref, bemb_ref, wn_ref, bn_ref, h_ref, pd_ref, ps_ref):
    h = jnp.dot(x_ref[...], wemb_ref[...], preferred_element_type=_F32) + bemb_ref[...]
    h_ref[...] = h
    p = jnp.dot(h, wn_ref[...], preferred_element_type=_F32) + bn_ref[...]
    pd_ref[...] = p[:, :2 * A]
    ps_ref[...] = p[:, 2 * A:]


def _embed_project(x, wembT, bemb, wn, bn):
    return pl.pallas_call(
        _emb_body,
        out_shape=(
            jax.ShapeDtypeStruct((N, A), _F32),
            jax.ShapeDtypeStruct((N, 2 * A), _F32),
            jax.ShapeDtypeStruct((N, 2 * A), _F32),
        ),
    )(x, wembT, bemb, wn, bn)


_GAMMA = (1.0 / (8.0 / (EF - 1)) ** 2)
_STEP = 8.0 / (EF - 1)

BLK_E = 4000


def _edge_body(d_ref, zd_ref, zs_ref, we_ref, out_ref):
    d = d_ref[...]
    centers = lax.broadcasted_iota(jnp.int32, (1, EF), 1).astype(_F32) * _STEP
    ea = jnp.exp(-_GAMMA * (d - centers) ** 2)
    t = zd_ref[...] + zs_ref[...]
    t = t + jnp.dot(ea, we_ref[...], preferred_element_type=_F32)
    a = t[:, :A]
    b = t[:, A:]
    out_ref[...] = jax.nn.sigmoid(a) * jax.nn.softplus(b)


def _edge_stage(d, zd, zs, we):
    grid = (E // BLK_E,)
    return pl.pallas_call(
        _edge_body,
        grid=grid,
        in_specs=[
            pl.BlockSpec((BLK_E, 1), lambda i: (i, 0)),
            pl.BlockSpec((BLK_E, 2 * A), lambda i: (i, 0)),
            pl.BlockSpec((BLK_E, 2 * A), lambda i: (i, 0)),
            pl.BlockSpec((EF, 2 * A), lambda i: (0, 0)),
        ],
        out_specs=pl.BlockSpec((BLK_E, A), lambda i: (i, 0)),
        out_shape=jax.ShapeDtypeStruct((E, A), _F32),
    )(d, zd, zs, we)


def _bn_proj_body(parts_ref, h_ref, g_ref, be_ref, wn_ref, bn_ref,
                  hout_ref, pd_ref, ps_ref):
    t = parts_ref[...] + h_ref[...]
    mu = jnp.mean(t, axis=0, keepdims=True)
    var = jnp.mean((t - mu) ** 2, axis=0, keepdims=True)
    hn = g_ref[...] * (t - mu) / jnp.sqrt(var + 1e-5) + be_ref[...]
    hout_ref[...] = hn
    p = jnp.dot(hn, wn_ref[...], preferred_element_type=_F32) + bn_ref[...]
    pd_ref[...] = p[:, :2 * A]
    ps_ref[...] = p[:, 2 * A:]


def _bn_project(parts, h, g, be, wn, bn):
    return pl.pallas_call(
        _bn_proj_body,
        out_shape=(
            jax.ShapeDtypeStruct((N, A), _F32),
            jax.ShapeDtypeStruct((N, 2 * A), _F32),
            jax.ShapeDtypeStruct((N, 2 * A), _F32),
        ),
    )(parts, h, g, be, wn, bn)


def _head_body(parts_ref, h_ref, g_ref, be_ref, batch_ref,
               wfc_ref, bfc_ref, wh0_ref, bh0_ref, wh1_ref, bh1_ref,
               wout_ref, bout_ref, out_ref):
    t = parts_ref[...] + h_ref[...]
    mu = jnp.mean(t, axis=0, keepdims=True)
    var = jnp.mean((t - mu) ** 2, axis=0, keepdims=True)
    hn = g_ref[...] * (t - mu) / jnp.sqrt(var + 1e-5) + be_ref[...]

    seg = lax.broadcasted_iota(jnp.int32, (1, G), 1)
    mask = (batch_ref[...] == seg).astype(_F32)
    sums = lax.dot_general(mask, hn, (((0,), (0,)), ((), ())),
                           preferred_element_type=_F32)
    ones = jnp.ones((N, 1), _F32)
    cnt = lax.dot_general(mask, ones, (((0,), (0,)), ((), ())),
                          preferred_element_type=_F32)
    pooled = sums / jnp.maximum(cnt, 1.0)

    o = jax.nn.softplus(pooled)
    o = jax.nn.softplus(jnp.dot(o, wfc_ref[...], preferred_element_type=_F32) + bfc_ref[...])
    o = jax.nn.softplus(jnp.dot(o, wh0_ref[...], preferred_element_type=_F32) + bh0_ref[...])
    o = jax.nn.softplus(jnp.dot(o, wh1_ref[...], preferred_element_type=_F32) + bh1_ref[...])
    out_ref[...] = jnp.dot(o, wout_ref[...], preferred_element_type=_F32) + bout_ref[...]


def _head(parts, h, g, be, batch2, wfcT, bfc, wh0T, bh0, wh1T, bh1, woutT, bout):
    return pl.pallas_call(
        _head_body,
        out_shape=jax.ShapeDtypeStruct((G, 1), _F32),
    )(parts, h, g, be, batch2, wfcT, bfc, wh0T, bh0, wh1T, bh1, woutT, bout)


def _gather_body(pd_hbm, ps_hbm, dst_hbm, src_hbm, zd_hbm, zs_hbm,
                 idx_d, idx_s, bufd, bufs, sem, sem2):
    wid = lax.axis_index("s") * NC + lax.axis_index("c")
    e_base = wid * EPW

    def chunk(ci, carry):
        e0 = e_base + ci * C
        pltpu.sync_copy(dst_hbm.at[pl.ds(e0, C)], idx_d)
        pltpu.sync_copy(src_hbm.at[pl.ds(e0, C)], idx_s)
        cpd = pltpu.async_copy(pd_hbm.at[idx_d], bufd, sem)
        cps = pltpu.async_copy(ps_hbm.at[idx_s], bufs, sem2)
        cpd.wait()
        cps.wait()
        pltpu.sync_copy(bufd, zd_hbm.at[pl.ds(e0, C)])
        pltpu.sync_copy(bufs, zs_hbm.at[pl.ds(e0, C)])
        return carry

    lax.fori_loop(0, NCH, chunk, 0)


@functools.cache
def _sc_gather_kernel():
    mesh = plsc.VectorSubcoreMesh(core_axis_name="c", subcore_axis_name="s",
                                  num_cores=NC, num_subcores=NS)
    return pl.kernel(
        _gather_body,
        out_type=(
            jax.ShapeDtypeStruct((E, 2 * A), _F32),
            jax.ShapeDtypeStruct((E, 2 * A), _F32),
        ),
        mesh=mesh,
        scratch_types=[
            pltpu.VMEM((C,), jnp.int32),
            pltpu.VMEM((C,), jnp.int32),
            pltpu.VMEM((C, 2 * A), _F32),
            pltpu.VMEM((C, 2 * A), _F32),
            pltpu.SemaphoreType.DMA,
            pltpu.SemaphoreType.DMA,
        ],
    )


def _sc_gather(pd, ps, dst2, src2):
    return _sc_gather_kernel()(pd, ps, dst2, src2)


def kernel(x, edge_index, edge_attr, batch, W_emb, b_emb,
           Wf0, bf0, Ws0, bs0, g0, be0,
           Wf1, bf1, Ws1, bs1, g1, be1,
           Wf2, bf2, Ws2, bs2, g2, be2,
           W_fc, b_fc, Wh0, bh0, Wh1, bh1, W_out, b_out):
    dst2 = edge_index[1].reshape(E)
    src2 = edge_index[0].reshape(E)
    d2 = edge_attr.reshape(E, 1)
    batch2 = batch.reshape(N, 1)

    def node_w(Wf, bf, Ws, bs):
        wn = jnp.concatenate(
            [Wf[:, :A].T, Ws[:, :A].T, Wf[:, A:2 * A].T, Ws[:, A:2 * A].T], axis=1)
        bn = jnp.concatenate([bf, bs, jnp.zeros((2 * A,), _F32)]).reshape(1, 4 * A)
        we = jnp.concatenate([Wf[:, 2 * A:].T, Ws[:, 2 * A:].T], axis=1)
        return wn, bn, we

    wn0, bn0, we0 = node_w(Wf0, bf0, Ws0, bs0)
    wn1, bn1, we1 = node_w(Wf1, bf1, Ws1, bs1)
    wn2, bn2, we2 = node_w(Wf2, bf2, Ws2, bs2)

    h, pd, ps = _embed_project(x, W_emb.T, b_emb.reshape(1, A), wn0, bn0)

    lw = [(we0, g0, be0, wn1, bn1), (we1, g1, be1, wn2, bn2)]
    for we, g, be, wn_n, bn_n in lw:
        zd, zs = _sc_gather(pd, ps, dst2, src2)
        contrib = _edge_stage(d2, zd, zs, we)
        parts = jax.ops.segment_sum(contrib, dst2, num_segments=N)
        h, pd, ps = _bn_project(parts, h, g.reshape(1, A), be.reshape(1, A), wn_n, bn_n)

    zd, zs = _sc_gather(pd, ps, dst2, src2)
    contrib = _edge_stage(d2, zd, zs, we2)
    parts = jax.ops.segment_sum(contrib, dst2, num_segments=N)

    out = _head(parts, h, g2.reshape(1, A), be2.reshape(1, A), batch2,
                W_fc.T, b_fc.reshape(1, H),
                Wh0.T, bh0.reshape(1, H), Wh1.T, bh1.reshape(1, H),
                W_out.T, b_out.reshape(1, 1))
    return out
